# baseline (speedup 1.0000x reference)
import jax
import jax.numpy as jnp
from jax import lax
from jax.experimental import pallas as pl
from jax.experimental.pallas import tpu as pltpu
from jax.experimental.pallas import tpu_sc as plsc

_ROWS = 256
_NC = 2
_NS = 16
_L = 16
_NW = _NC * _NS


def _sc_select_weights(rew_flat, wbc, base0, ntok):
    tok_per_w = ntok // _NW
    mesh = plsc.VectorSubcoreMesh(core_axis_name="c", subcore_axis_name="s")

    def body(rew_hbm, wbc_hbm, wa_hbm, wb_hbm, idx_v, wbc_v, wa_v, wb_v):
        wid = lax.axis_index("s") * _NC + lax.axis_index("c")
        base = wid * tok_per_w
        pltpu.sync_copy(wbc_hbm, wbc_v)
        pltpu.sync_copy(rew_hbm.at[pl.ds(base0 + base, tok_per_w)], idx_v)
        w00v = wbc_v[0, :]
        w01v = wbc_v[1, :]
        w10v = wbc_v[2, :]
        w11v = wbc_v[3, :]
        for c in range(tok_per_w // _L):
            r16 = idx_v[pl.ds(c * _L, _L)]
            m = r16 == 0
            wa_v[pl.ds(c * _L, _L)] = jnp.where(m, w00v, w10v)
            wb_v[pl.ds(c * _L, _L)] = jnp.where(m, w01v, w11v)
        pltpu.sync_copy(wa_v, wa_hbm.at[pl.ds(base, tok_per_w)])
        pltpu.sync_copy(wb_v, wb_hbm.at[pl.ds(base, tok_per_w)])

    f = pl.kernel(
        body,
        out_type=[
            jax.ShapeDtypeStruct((ntok,), jnp.float32),
            jax.ShapeDtypeStruct((ntok,), jnp.float32),
        ],
        mesh=mesh,
        scratch_types=[
            pltpu.VMEM((tok_per_w,), jnp.int32),
            pltpu.VMEM((4, _L), jnp.float32),
            pltpu.VMEM((tok_per_w,), jnp.float32),
            pltpu.VMEM((tok_per_w,), jnp.float32),
        ],
    )
    return f(rew_flat, wbc)


def _blend_inline_body(w_ref, idx_ref, x_ref, y_ref, o_ref):
    r = idx_ref[:, :]
    sel = r == 0
    w0 = jnp.where(sel, w_ref[0, 0], w_ref[1, 0])
    w1 = jnp.where(sel, w_ref[0, 1], w_ref[1, 1])
    o_ref[:, :] = x_ref[:, :] * w0 + y_ref[:, :] * w1


def _blend_weighted_body(prev_ref, wa_ref, wb_ref, x_ref, y_ref, o_ref):
    del prev_ref
    o_ref[:, :] = x_ref[:, :] * wa_ref[:, :] + y_ref[:, :] * wb_ref[:, :]


def kernel(X, Y, reward, W):
    B, S, D = X.shape
    N = B * S
    half = N // 2
    hblk = half // _ROWS
    x2 = X.reshape(N, D)
    y2 = Y.reshape(N, D)
    rew_flat = reward.reshape(N)
    idx = reward.reshape(N, 1)
    wbc = jnp.broadcast_to(
        W.reshape(2, 2, 1), (2, 2, _L)
    ).reshape(4, _L)

    wa, wb = _sc_select_weights(rew_flat, wbc, half, N - half)

    out1 = pl.pallas_call(
        _blend_inline_body,
        grid=(hblk,),
        in_specs=[
            pl.BlockSpec(memory_space=pltpu.SMEM),
            pl.BlockSpec((_ROWS, 1), lambda i: (i, 0)),
            pl.BlockSpec((_ROWS, D), lambda i: (i, 0)),
            pl.BlockSpec((_ROWS, D), lambda i: (i, 0)),
        ],
        out_specs=pl.BlockSpec((_ROWS, D), lambda i: (i, 0)),
        out_shape=jax.ShapeDtypeStruct((N, D), jnp.float32),
        compiler_params=pltpu.CompilerParams(
            dimension_semantics=("parallel",),
        ),
    )(W, idx, x2, y2)

    out = pl.pallas_call(
        _blend_weighted_body,
        grid=(hblk,),
        in_specs=[
            pl.BlockSpec(memory_space=pl.ANY),
            pl.BlockSpec((_ROWS, 1), lambda i: (i, 0)),
            pl.BlockSpec((_ROWS, 1), lambda i: (i, 0)),
            pl.BlockSpec((_ROWS, D), lambda i: (i + hblk, 0)),
            pl.BlockSpec((_ROWS, D), lambda i: (i + hblk, 0)),
        ],
        out_specs=pl.BlockSpec((_ROWS, D), lambda i: (i + hblk, 0)),
        out_shape=jax.ShapeDtypeStruct((N, D), jnp.float32),
        input_output_aliases={0: 0},
        compiler_params=pltpu.CompilerParams(
            dimension_semantics=("parallel",),
        ),
    )(out1, wa.reshape(half, 1), wb.reshape(half, 1), x2, y2)
    return out.reshape(B, S, D)


# speedup vs baseline: 1.3521x; 1.3521x over previous
import jax
import jax.numpy as jnp
from jax.experimental import pallas as pl
from jax.experimental.pallas import tpu as pltpu

_ROWS = 256


def _blend_body(w_ref, idx_ref, x_ref, y_ref, o_ref):
    r = idx_ref[:, :]
    sel = r == 0
    w0 = jnp.where(sel, w_ref[0, 0], w_ref[1, 0])
    w1 = jnp.where(sel, w_ref[0, 1], w_ref[1, 1])
    o_ref[:, :] = x_ref[:, :] * w0 + y_ref[:, :] * w1


def kernel(X, Y, reward, W):
    B, S, D = X.shape
    N = B * S
    x2 = X.reshape(N, D)
    y2 = Y.reshape(N, D)
    idx = reward.reshape(N, 1)

    grid = (N // _ROWS,)
    out = pl.pallas_call(
        _blend_body,
        grid=grid,
        in_specs=[
            pl.BlockSpec(memory_space=pltpu.SMEM),
            pl.BlockSpec((_ROWS, 1), lambda i: (i, 0)),
            pl.BlockSpec((_ROWS, D), lambda i: (i, 0)),
            pl.BlockSpec((_ROWS, D), lambda i: (i, 0)),
        ],
        out_specs=pl.BlockSpec((_ROWS, D), lambda i: (i, 0)),
        out_shape=jax.ShapeDtypeStruct((N, D), jnp.float32),
        compiler_params=pltpu.CompilerParams(
            dimension_semantics=("parallel",),
        ),
    )(W, idx, x2, y2)
    return out.reshape(B, S, D)
